# baseline (speedup 1.0000x reference)
import functools

import jax
import jax.numpy as jnp
from jax import lax
from jax.experimental import pallas as pl
from jax.experimental.pallas import tpu as pltpu
from jax.experimental.pallas import tpu_sc as plsc

N_TOK = 32768
DIM = 128
NSEG = 16
TILE = 16384
GRID = N_TOK // TILE
NWORK = 32
CHUNK = N_TOK // NWORK


def _leaky(x):
    return jnp.maximum(x, 0.2 * x)


def _smem_to_col(ref, n, offset=0):
    sub = lax.broadcasted_iota(jnp.int32, (n, 1), 0)
    col = jnp.zeros((n, 1), ref.dtype)
    for s in range(n):
        col = jnp.where(sub == s, ref[s + offset], col)
    return col


def _mlp_kernel(b1_ref, b2_ref, x_ref, w1_ref, w2_ref, out_ref):
    h1 = _leaky(jnp.dot(x_ref[...].astype(jnp.bfloat16),
                        w1_ref[...].astype(jnp.bfloat16),
                        preferred_element_type=jnp.float32)
                + _smem_to_col(b1_ref, 32).reshape(1, 32))
    out_ref[...] = _leaky(jnp.dot(h1.astype(jnp.bfloat16),
                                  w2_ref[...].astype(jnp.bfloat16),
                                  preferred_element_type=jnp.float32)
                          + _smem_to_col(b2_ref, NSEG).reshape(1, NSEG))


SUB = 512
NSUB = CHUNK // SUB


def _sc_segsum(h2_hbm, lo_hbm, hi_hbm, out_hbm, buf, lo_v, hi_v, acc_v):
    wid = lax.axis_index("c") * 16 + lax.axis_index("s")
    base = wid * CHUNK
    pltpu.sync_copy(lo_hbm, lo_v)
    pltpu.sync_copy(hi_hbm, hi_v)
    lo_vec = lo_v[...]
    hi_vec = hi_v[...]
    for s in range(NSEG):
        acc_v[s, :] = jnp.zeros((16,), jnp.float32)
    for j in range(NSUB):
        sub_base = base + j * SUB
        pltpu.sync_copy(h2_hbm.at[pl.ds(sub_base, SUB)], buf)
        for s in range(NSEG):
            a = jnp.maximum(lo_vec[s], sub_base)
            b = jnp.minimum(hi_vec[s], sub_base + SUB)

            def body(t, acc):
                return acc + buf[t - sub_base]

            acc = lax.fori_loop(a, b, body, jnp.zeros((16,), jnp.float32),
                                unroll=False)
            acc_v[s, :] = acc_v[s, :] + acc
    pltpu.sync_copy(acc_v, out_hbm.at[wid])


def _merge_kernel(ptr_ref, part_ref, w3_ref, b3_ref, out_ref):
    acc = jnp.sum(part_ref[...], axis=0)
    lo = _smem_to_col(ptr_ref, NSEG)
    hi = _smem_to_col(ptr_ref, NSEG, offset=1)
    cnt = (hi - lo).astype(jnp.float32)
    out_ref[...] = (
        jnp.dot(acc, w3_ref[...], preferred_element_type=jnp.float32)
        + cnt * b3_ref[...]
    )


def kernel(h_dag, obs_ptr, W1, b1, W2, b2, W3, b3):
    const = lambda i, *refs: (0, 0)
    grid_spec = pltpu.PrefetchScalarGridSpec(
        num_scalar_prefetch=2,
        grid=(GRID,),
        in_specs=[
            pl.BlockSpec((TILE, DIM), lambda i, *refs: (i, 0)),
            pl.BlockSpec((DIM, 32), const),
            pl.BlockSpec((32, 16), const),
        ],
        out_specs=pl.BlockSpec((TILE, NSEG), lambda i, *refs: (i, 0)),
        scratch_shapes=[],
    )
    h2 = pl.pallas_call(
        _mlp_kernel,
        grid_spec=grid_spec,
        out_shape=jax.ShapeDtypeStruct((N_TOK, NSEG), jnp.float32),
        compiler_params=pltpu.CompilerParams(
            dimension_semantics=("arbitrary",),
        ),
    )(b1, b2, h_dag, W1, W2)

    mesh = plsc.VectorSubcoreMesh(core_axis_name="c", subcore_axis_name="s")
    sc_call = functools.partial(
        pl.kernel, mesh=mesh,
        out_type=jax.ShapeDtypeStruct((NWORK, NSEG, 16), jnp.float32),
        scratch_types=[
            pltpu.VMEM((512, NSEG), jnp.float32),
            pltpu.VMEM((NSEG,), jnp.int32),
            pltpu.VMEM((NSEG,), jnp.int32),
            pltpu.VMEM((NSEG, 16), jnp.float32),
        ],
    )
    partials = sc_call(_sc_segsum)(h2, obs_ptr[:NSEG], obs_ptr[1:NSEG + 1])

    merge_spec = pltpu.PrefetchScalarGridSpec(
        num_scalar_prefetch=1,
        grid=(1,),
        in_specs=[
            pl.BlockSpec((NWORK, NSEG, 16), lambda i, *refs: (0, 0, 0)),
            pl.BlockSpec((16, DIM), const),
            pl.BlockSpec((1, DIM), const),
        ],
        out_specs=pl.BlockSpec((NSEG, DIM), const),
        scratch_shapes=[],
    )
    out = pl.pallas_call(
        _merge_kernel,
        grid_spec=merge_spec,
        out_shape=jax.ShapeDtypeStruct((NSEG, DIM), jnp.float32),
    )(obs_ptr, partials, W3, b3.reshape(1, DIM))
    return out


# speedup vs baseline: 4.0322x; 4.0322x over previous
import jax
import jax.numpy as jnp
from jax.experimental import pallas as pl
from jax.experimental.pallas import tpu as pltpu

N_TOK = 32768
DIM = 128
NSEG = 16
TILE = 8192
GRID = N_TOK // (2 * TILE)


def _leaky(x):
    return jnp.maximum(x, 0.2 * x)


def _smem_to_col(ref, n, offset=0):
    sub = jax.lax.broadcasted_iota(jnp.int32, (n, 1), 0)
    col = jnp.zeros((n, 1), ref.dtype)
    for s in range(n):
        col = jnp.where(sub == s, ref[s + offset], col)
    return col


def _mlp_seg(x, pid_off, w1, w2, b1c, b2c, lo, hi):
    h1t = _leaky(
        jax.lax.dot_general(w1.astype(jnp.bfloat16),
                            x.astype(jnp.bfloat16),
                            (((0,), (1,)), ((), ())),
                            preferred_element_type=jnp.float32)
        + b1c)
    h2t = _leaky(
        jax.lax.dot_general(w2.astype(jnp.bfloat16),
                            h1t.astype(jnp.bfloat16),
                            (((0,), (0,)), ((), ())),
                            preferred_element_type=jnp.float32)
        + b2c)
    cols = jax.lax.broadcasted_iota(jnp.int32, (NSEG, TILE), 1) + pid_off
    m = jnp.logical_and(cols >= lo, cols < hi)
    return jax.lax.dot_general(
        m.astype(jnp.bfloat16), h2t.astype(jnp.bfloat16),
        (((1,), (1,)), ((), ())),
        preferred_element_type=jnp.float32)


def _fused_kernel(ptr_ref, b1_ref, b2_ref, xa_ref, xb_ref, w1_ref, w2_ref,
                  w3_ref, b3_ref, out_ref, acc_ref, lo_ref, hi_ref,
                  b1c_ref, b2c_ref):
    pid = pl.program_id(0)

    @pl.when(pid == 0)
    def _init():
        acc_ref[...] = jnp.zeros_like(acc_ref)
        lo_ref[...] = _smem_to_col(ptr_ref, NSEG)
        hi_ref[...] = _smem_to_col(ptr_ref, NSEG, offset=1)
        b1c_ref[...] = _smem_to_col(b1_ref, 32)
        b2c_ref[...] = _smem_to_col(b2_ref, NSEG)

    w1, w2 = w1_ref[...], w2_ref[...]
    b1c, b2c = b1c_ref[...], b2c_ref[...]
    lo, hi = lo_ref[...], hi_ref[...]
    acc_ref[...] += (
        _mlp_seg(xa_ref[...], pid * TILE, w1, w2, b1c, b2c, lo, hi)
        + _mlp_seg(xb_ref[...], (GRID + pid) * TILE, w1, w2, b1c, b2c, lo, hi))

    @pl.when(pid == GRID - 1)
    def _finish():
        cnt = (hi_ref[...] - lo_ref[...]).astype(jnp.float32)
        out_ref[...] = (
            jnp.dot(acc_ref[...], w3_ref[...], preferred_element_type=jnp.float32)
            + cnt * b3_ref[...]
        )


def kernel(h_dag, obs_ptr, W1, b1, W2, b2, W3, b3):
    const = lambda i, *refs: (0, 0)
    grid_spec = pltpu.PrefetchScalarGridSpec(
        num_scalar_prefetch=3,
        grid=(GRID,),
        in_specs=[
            pl.BlockSpec((TILE, DIM), lambda i, *refs: (i, 0)),
            pl.BlockSpec((TILE, DIM), lambda i, *refs: (i + GRID, 0)),
            pl.BlockSpec((DIM, 32), const),
            pl.BlockSpec((32, 16), const),
            pl.BlockSpec((16, DIM), const),
            pl.BlockSpec((1, DIM), const),
        ],
        out_specs=pl.BlockSpec((NSEG, DIM), const),
        scratch_shapes=[
            pltpu.VMEM((NSEG, 16), jnp.float32),
            pltpu.VMEM((NSEG, 1), jnp.int32),
            pltpu.VMEM((NSEG, 1), jnp.int32),
            pltpu.VMEM((32, 1), jnp.float32),
            pltpu.VMEM((NSEG, 1), jnp.float32),
        ],
    )
    out = pl.pallas_call(
        _fused_kernel,
        grid_spec=grid_spec,
        out_shape=jax.ShapeDtypeStruct((NSEG, DIM), jnp.float32),
        compiler_params=pltpu.CompilerParams(
            dimension_semantics=("arbitrary",),
        ),
    )(obs_ptr, b1, b2, h_dag, h_dag, W1, W2, W3, b3.reshape(1, DIM))
    return out


# speedup vs baseline: 4.0912x; 1.0146x over previous
import jax
import jax.numpy as jnp
from jax.experimental import pallas as pl
from jax.experimental.pallas import tpu as pltpu

N_TOK = 32768
DIM = 128
NSEG = 16
TILE = 16384
GRID = N_TOK // TILE


def _leaky(x):
    return jnp.maximum(x, 0.2 * x)


def _smem_to_col(ref, n, offset=0):
    sub = jax.lax.broadcasted_iota(jnp.int32, (n, 1), 0)
    col = jnp.zeros((n, 1), ref.dtype)
    for s in range(n):
        col = jnp.where(sub == s, ref[s + offset], col)
    return col


def _fused_kernel(ptr_ref, b1_ref, b2_ref, x_ref, w1_ref, w2_ref,
                  w3_ref, b3_ref, out_ref, acc_ref, lo_ref, hi_ref,
                  b1c_ref, b2c_ref):
    pid = pl.program_id(0)

    @pl.when(pid == 0)
    def _init():
        acc_ref[...] = jnp.zeros_like(acc_ref)
        lo_ref[...] = _smem_to_col(ptr_ref, NSEG)
        hi_ref[...] = _smem_to_col(ptr_ref, NSEG, offset=1)
        b1c_ref[...] = _smem_to_col(b1_ref, 32)
        b2c_ref[...] = _smem_to_col(b2_ref, NSEG)

    h1t = _leaky(
        jax.lax.dot_general(w1_ref[...].astype(jnp.bfloat16),
                            x_ref[...].astype(jnp.bfloat16),
                            (((0,), (1,)), ((), ())),
                            preferred_element_type=jnp.float32)
        + b1c_ref[...])
    h2t = _leaky(
        jax.lax.dot_general(w2_ref[...].astype(jnp.bfloat16),
                            h1t.astype(jnp.bfloat16),
                            (((0,), (0,)), ((), ())),
                            preferred_element_type=jnp.float32)
        + b2c_ref[...])

    cols = jax.lax.broadcasted_iota(jnp.int32, (NSEG, TILE), 1) + pid * TILE
    m = jnp.logical_and(cols >= lo_ref[...], cols < hi_ref[...])

    acc_ref[...] += jax.lax.dot_general(
        m.astype(jnp.bfloat16), h2t.astype(jnp.bfloat16),
        (((1,), (1,)), ((), ())),
        preferred_element_type=jnp.float32)

    @pl.when(pid == GRID - 1)
    def _finish():
        cnt = (hi_ref[...] - lo_ref[...]).astype(jnp.float32)
        out_ref[...] = (
            jnp.dot(acc_ref[...], w3_ref[...], preferred_element_type=jnp.float32)
            + cnt * b3_ref[...]
        )


def kernel(h_dag, obs_ptr, W1, b1, W2, b2, W3, b3):
    const = lambda i, *refs: (0, 0)
    grid_spec = pltpu.PrefetchScalarGridSpec(
        num_scalar_prefetch=3,
        grid=(GRID,),
        in_specs=[
            pl.BlockSpec((TILE, DIM), lambda i, *refs: (i, 0)),
            pl.BlockSpec((DIM, 32), const),
            pl.BlockSpec((32, 16), const),
            pl.BlockSpec((16, DIM), const),
            pl.BlockSpec((1, DIM), const),
        ],
        out_specs=pl.BlockSpec((NSEG, DIM), const),
        scratch_shapes=[
            pltpu.VMEM((NSEG, 16), jnp.float32),
            pltpu.VMEM((NSEG, 1), jnp.int32),
            pltpu.VMEM((NSEG, 1), jnp.int32),
            pltpu.VMEM((32, 1), jnp.float32),
            pltpu.VMEM((NSEG, 1), jnp.float32),
        ],
    )
    out = pl.pallas_call(
        _fused_kernel,
        grid_spec=grid_spec,
        out_shape=jax.ShapeDtypeStruct((NSEG, DIM), jnp.float32),
        compiler_params=pltpu.CompilerParams(
            dimension_semantics=("arbitrary",),
        ),
    )(obs_ptr, b1, b2, h_dag, W1, W2, W3, b3.reshape(1, DIM))
    return out
